# Initial kernel scaffold; baseline (speedup 1.0000x reference)
import jax
import jax.numpy as jnp
from jax.experimental import pallas as pl


def kernel(x):
    raise NotImplementedError("write your pallas kernel here")


# speedup vs baseline: 4.6670x; 4.6670x over previous
import jax
import jax.numpy as jnp
from jax.experimental import pallas as pl
from jax.experimental.pallas import tpu as pltpu

_BC = 8


def _pool_body(x_ref, o_ref):
    x = x_ref[...]
    rm = jnp.maximum(x[:, :-1, :], x[:, 1:, :])
    o_ref[...] = jnp.maximum(rm[:, :, :-1], rm[:, :, 1:])


def kernel(x):
    N, C, H, W = x.shape
    nc = N * C
    xf = x.reshape(nc, H, W)
    out = pl.pallas_call(
        _pool_body,
        grid=(nc // _BC,),
        in_specs=[pl.BlockSpec((_BC, H, W), lambda i: (i, 0, 0))],
        out_specs=pl.BlockSpec((_BC, H - 1, W - 1), lambda i: (i, 0, 0)),
        out_shape=jax.ShapeDtypeStruct((nc, H - 1, W - 1), x.dtype),
        compiler_params=pltpu.CompilerParams(
            dimension_semantics=("parallel",),
        ),
    )(xf)
    return out.reshape(N, C, H - 1, W - 1)


# speedup vs baseline: 5.1687x; 1.1075x over previous
import jax
import jax.numpy as jnp
from jax.experimental import pallas as pl
from jax.experimental.pallas import tpu as pltpu

_BC = 16


def _pool_body(x_ref, o_ref):
    x = x_ref[...]
    rm = jnp.maximum(x[:, :-1, :], x[:, 1:, :])
    o_ref[...] = jnp.maximum(rm[:, :, :-1], rm[:, :, 1:])


def kernel(x):
    N, C, H, W = x.shape
    nc = N * C
    xf = x.reshape(nc, H, W)
    out = pl.pallas_call(
        _pool_body,
        grid=(nc // _BC,),
        in_specs=[pl.BlockSpec((_BC, H, W), lambda i: (i, 0, 0))],
        out_specs=pl.BlockSpec((_BC, H - 1, W - 1), lambda i: (i, 0, 0)),
        out_shape=jax.ShapeDtypeStruct((nc, H - 1, W - 1), x.dtype),
        compiler_params=pltpu.CompilerParams(
            dimension_semantics=("parallel",),
        ),
    )(xf)
    return out.reshape(N, C, H - 1, W - 1)
